# baseline (speedup 1.0000x reference)
import jax
import jax.numpy as jnp
from jax import lax
from jax.experimental import pallas as pl
from jax.experimental.pallas import tpu as pltpu
from jax.experimental.pallas import tpu_sc as plsc

N = 10000
NP = 10240
E = 320000
HEADS = 4
NEG = 0.01

K = 256
NCH = E // K
SUBC = 16
CORES = 2
GI = -(-NCH // SUBC)
NPT = NP // SUBC
NROW = 128
NSUB = NPT // NROW

_MESH = plsc.VectorSubcoreMesh(core_axis_name="c", subcore_axis_name="s",
                               num_cores=CORES, num_subcores=SUBC)
_SC_PARAMS = pltpu.CompilerParams(needs_layout_passes=False,
                                  use_tc_tiling_on_sc=False)


BR = 2048


def _tc0_body(x_ref, w_ref, b_ref, wa_ref, ba_ref, ft_ref, a12_ref):
    ft = jnp.dot(x_ref[...], w_ref[0], preferred_element_type=jnp.float32)
    ft = ft + b_ref[0]
    ft_ref[...] = ft
    a12 = jnp.dot(ft, wa_ref[0], preferred_element_type=jnp.float32)
    a12_ref[0] = a12 + ba_ref[0]


def _tc1_body(l0_ref, l1_ref, l2_ref, l3_ref, w_ref, b_ref, wa_ref, ba_ref,
              wres_ref, bres_ref, ft_ref, a12_ref, res_ref):
    lastc = jnp.concatenate(
        [l0_ref[...], l1_ref[...], l2_ref[...], l3_ref[...]], axis=1)
    ft = jnp.dot(lastc, w_ref[0], preferred_element_type=jnp.float32)
    ft = ft + b_ref[0]
    ft_ref[...] = ft
    a12 = jnp.dot(ft, wa_ref[0], preferred_element_type=jnp.float32)
    a12_ref[0] = a12 + ba_ref[0]
    res = jnp.dot(lastc, wres_ref[0], preferred_element_type=jnp.float32)
    res_ref[...] = res + bres_ref[0]


def _elu16(x):
    return jnp.where(x > 0, x, jnp.exp(x) - 1.0)


def _edge_pass(hh, ss, ftflat_hbm, a12_v, ed_hbm, zrow_hbm, zdnm_hbm,
               edb, srcg, dstg, rows_v, dnm_v, semg, seme, sems,
               accum_sh, denom_sh):
    iota16 = lax.iota(jnp.int32, 16)
    col0 = jnp.zeros((16,), jnp.int32)
    ftoff = hh * NP
    T = jnp.where(ss < NCH - (NCH // SUBC) * SUBC,
                  NCH // SUBC + 1, NCH // SUBC)

    def score(g, b):
        for v in range(K // 16):
            j, o = v // 8, (v % 8) * 16
            src16 = edb[b, j, pl.ds(o, 16)]
            dst16 = edb[b, 2 + j, pl.ds(o, 16)]
            a1 = plsc.load_gather(a12_v, [dst16 * 2])
            a2 = plsc.load_gather(a12_v, [src16 * 2 + 1])
            s = a1 + a2
            s = jnp.where(s > 0, s, NEG * s)
            ex = jnp.exp(s)
            rowi = iota16 + (v * 16)
            plsc.store_scatter(dnm_v.at[b], [rowi, col0], ex)
            srcg[b, j, pl.ds(o, 16)] = src16 + ftoff
            dstg[b, j, pl.ds(o, 16)] = dst16

    def fire_edge(g, b):
        pltpu.async_copy(ed_hbm.at[ss + g * SUBC], edb.at[b], seme)

    def wait_edge(b):
        pltpu.make_async_copy(ed_hbm.at[0], edb.at[b], seme).wait()

    def fire_gather(b):
        for j in range(2):
            pltpu.async_copy(ftflat_hbm.at[srcg.at[b, j]],
                             rows_v.at[b, pl.ds(j * 128, 128)], semg)

    def wait_gather(b):
        for j in range(2):
            pltpu.make_async_copy(zrow_hbm.at[pl.ds(0, 128)],
                                  rows_v.at[b, pl.ds(j * 128, 128)],
                                  semg).wait()

    def fire_scatter(b):
        for j in range(2):
            pltpu.async_copy(rows_v.at[b, pl.ds(j * 128, 128)],
                             accum_sh.at[dstg.at[b, j]], sems, add=True)
            pltpu.async_copy(dnm_v.at[b, pl.ds(j * 128, 128)],
                             denom_sh.at[dstg.at[b, j]], sems, add=True)

    def wait_scatter(b):
        for j in range(2):
            pltpu.make_async_copy(zrow_hbm.at[pl.ds(0, 128)],
                                  rows_v.at[b, pl.ds(j * 128, 128)],
                                  sems).wait()
            pltpu.make_async_copy(zdnm_hbm.at[pl.ds(0, 128)],
                                  dnm_v.at[b, pl.ds(j * 128, 128)],
                                  sems).wait()

    def scale(b):
        @plsc.parallel_loop(0, K, 1, unroll=8)
        def scale_body(e):
            dr = dnm_v[b, e, :]
            w0 = jnp.full((16,), dr[0], jnp.float32)
            for jv in range(4):
                x = rows_v[b, e, pl.ds(jv * 16, 16)]
                rows_v[b, e, pl.ds(jv * 16, 16)] = x * w0

    pltpu.sync_copy(ed_hbm.at[ss], edb.at[0])
    score(0, 0)
    fire_gather(0)

    @pl.when(T > 1)
    def _():
        fire_edge(1, 1)

    def body(g, _):
        b = g % 2
        nb = 1 - b
        wait_gather(b)

        @pl.when(g + 1 < T)
        def _():
            wait_edge(nb)

        @pl.when(g >= 1)
        def _():
            wait_scatter(nb)

        @pl.when(g + 1 < T)
        def _():
            score(g + 1, nb)
            fire_gather(nb)

        @pl.when(g + 2 < T)
        def _():
            fire_edge(g + 2, b)

        scale(b)
        fire_scatter(b)
        return 0

    def guarded(g, c):
        @pl.when(g < T)
        def _():
            body(g, c)
        return 0

    lax.fori_loop(0, GI, guarded, 0)
    wait_scatter((T - 1) % 2)


def _zero_dnm(dnm_v):
    for b in range(2):
        @plsc.parallel_loop(0, K, 1, unroll=8)
        def zdn(i):
            dnm_v[b, i, :] = jnp.zeros((16,), jnp.float32)


def _sc0_body(ftflat_hbm, a12_hbm, ed_hbm, zrow_hbm, zdnm_hbm,
              out_hbm,
              a12_v, edb, srcg, dstg, rows_v, dnm_v, semg, seme, sems,
              accum_sh, denom_sh):
    cc = lax.axis_index("c")
    ss = lax.axis_index("s")
    n0 = ss * NPT
    _zero_dnm(dnm_v)
    for p in range(2):
        hh = 2 * cc + p
        pltpu.sync_copy(a12_hbm.at[hh], a12_v)
        pltpu.sync_copy(zrow_hbm.at[pl.ds(n0, NPT)],
                        accum_sh.at[pl.ds(n0, NPT)])
        pltpu.sync_copy(zdnm_hbm.at[pl.ds(n0, NPT)],
                        denom_sh.at[pl.ds(n0, NPT)])
        plsc.subcore_barrier()
        _edge_pass(hh, ss, ftflat_hbm, a12_v, ed_hbm, zrow_hbm, zdnm_hbm,
                   edb, srcg, dstg, rows_v, dnm_v, semg, seme, sems,
                   accum_sh, denom_sh)
        plsc.subcore_barrier()
        _zero_dnm(dnm_v)

        for t in range(NSUB):
            ns = n0 + t * NROW
            pltpu.sync_copy(accum_sh.at[pl.ds(ns, NROW)],
                            rows_v.at[0, pl.ds(0, NROW)])
            pltpu.sync_copy(denom_sh.at[pl.ds(ns, NROW)],
                            dnm_v.at[0, pl.ds(0, NROW)])

            @plsc.parallel_loop(0, NROW, 1, unroll=4)
            def norm_body(r):
                dr = dnm_v[0, r, :]
                d0 = jnp.maximum(jnp.full((16,), dr[0], jnp.float32), 1e-16)
                i0 = 1.0 / d0
                for jv in range(4):
                    x = rows_v[0, r, pl.ds(jv * 16, 16)] * i0
                    rows_v[0, r, pl.ds(jv * 16, 16)] = _elu16(x)
            pltpu.sync_copy(rows_v.at[0, pl.ds(0, NROW)],
                            out_hbm.at[pl.ds(hh * NP + ns, NROW)])
        _zero_dnm(dnm_v)
        plsc.subcore_barrier()


def _sc1_body(ftflat_hbm, a12_hbm, ed_hbm, zrow_hbm, zdnm_hbm,
              res_hbm, tp_hbm, out_hbm,
              a12_v, edb, flag_v, tb_v, psrc, pdst, pex,
              srcg2, dstb2, rows_v, dnm2, outb_v, semg, seme, sems,
              accum_sh, denom_sh):
    cc = lax.axis_index("c")
    ss = lax.axis_index("s")
    n0 = ss * NPT
    iota16 = lax.iota(jnp.int32, 16)
    col0 = jnp.zeros((16,), jnp.int32)
    T = jnp.where(ss < NCH - (NCH // SUBC) * SUBC,
                  NCH // SUBC + 1, NCH // SUBC)

    @plsc.parallel_loop(0, NP // 16, 1, unroll=8)
    def zf(i):
        flag_v[pl.ds(i * 16, 16)] = jnp.zeros((16,), jnp.int32)

    pltpu.sync_copy(tp_hbm, tb_v)
    ones16 = jnp.full((16,), 1, jnp.int32)
    for v in range(64):
        t16 = tb_v[pl.ds(v * 16, 16)]
        plsc.store_scatter(flag_v, [t16], ones16)

    @plsc.parallel_loop(0, 128, 1, unroll=8)
    def zd(i):
        dnm2[i, :] = jnp.zeros((16,), jnp.float32)

    def flush(ftoff):
        for v in range(8):
            srcg2[0, pl.ds(v * 16, 16)] = psrc[pl.ds(v * 16, 16)]
            dstb2[0, pl.ds(v * 16, 16)] = pdst[pl.ds(v * 16, 16)]
            exv = pex[pl.ds(v * 16, 16)]
            plsc.store_scatter(dnm2, [iota16 + v * 16, col0], exv)
        pltpu.async_copy(ftflat_hbm.at[srcg2.at[0]], rows_v.at[0], semg).wait()

        @plsc.parallel_loop(0, 128, 1, unroll=8)
        def scale_body(e):
            dr = dnm2[e, :]
            w0 = jnp.full((16,), dr[0], jnp.float32)
            for jv in range(4):
                x = rows_v[0, e, pl.ds(jv * 16, 16)]
                rows_v[0, e, pl.ds(jv * 16, 16)] = x * w0

        pltpu.async_copy(rows_v.at[0], accum_sh.at[dstb2.at[0]], sems,
                         add=True)
        pltpu.async_copy(dnm2, denom_sh.at[dstb2.at[0]], sems, add=True)
        pltpu.make_async_copy(zrow_hbm.at[pl.ds(0, 128)], rows_v.at[0],
                              sems).wait()
        pltpu.make_async_copy(zdnm_hbm.at[pl.ds(0, 128)], dnm2, sems).wait()

    def shift_pend():
        for v in range(16):
            psrc[pl.ds(v * 16, 16)] = psrc[pl.ds(128 + v * 16, 16)]
            pdst[pl.ds(v * 16, 16)] = pdst[pl.ds(128 + v * 16, 16)]
            pex[pl.ds(v * 16, 16)] = pex[pl.ds(128 + v * 16, 16)]

    for p in range(2):
        hh = 2 * cc + p
        ftoff = hh * NP
        pltpu.sync_copy(a12_hbm.at[hh], a12_v)
        pltpu.sync_copy(zrow_hbm.at[pl.ds(n0, NPT)],
                        accum_sh.at[pl.ds(n0, NPT)])
        pltpu.sync_copy(zdnm_hbm.at[pl.ds(n0, NPT)],
                        denom_sh.at[pl.ds(n0, NPT)])
        plsc.subcore_barrier()

        pltpu.sync_copy(ed_hbm.at[ss], edb.at[0])

        @pl.when(T > 1)
        def _():
            pltpu.async_copy(ed_hbm.at[ss + SUBC], edb.at[1], seme)

        def body(g, cnt):
            b = g % 2
            valid = g < T

            @pl.when((g >= 1) & (g < T))
            def _():
                pltpu.make_async_copy(ed_hbm.at[0], edb.at[b], seme).wait()

            for v in range(K // 16):
                j, o = v // 8, (v % 8) * 16
                src16 = edb[b, j, pl.ds(o, 16)]
                dst16 = edb[b, 2 + j, pl.ds(o, 16)]
                fl = plsc.load_gather(flag_v, [dst16])
                m = jnp.logical_and(fl > 0, valid)
                a1 = plsc.load_gather(a12_v, [dst16 * 2])
                a2 = plsc.load_gather(a12_v, [src16 * 2 + 1])
                s = a1 + a2
                s = jnp.where(s > 0, s, NEG * s)
                ex = jnp.exp(s)
                plsc.store_compressed(psrc.at[pl.ds(cnt, 16)],
                                      src16 + ftoff, mask=m)
                plsc.store_compressed(pdst.at[pl.ds(cnt, 16)], dst16, mask=m)
                plsc.store_compressed(pex.at[pl.ds(cnt, 16)], ex, mask=m)
                pc = plsc.all_reduce_population_count(m)
                cnt = cnt + pc[0]

            for _rep in range(2):
                @pl.when(cnt >= 128)
                def _():
                    flush(ftoff)
                    shift_pend()

                cnt = jnp.where(cnt >= 128, cnt - 128, cnt)

            nxt = jnp.minimum(ss + (g + 2) * SUBC, NCH - 1)

            @pl.when((g + 2 < T))
            def _():
                pltpu.async_copy(ed_hbm.at[nxt], edb.at[b], seme)

            return cnt

        cnt = lax.fori_loop(0, GI, body, 0)
        zero16f = jnp.zeros((16,), jnp.float32)
        off16 = jnp.full((16,), hh * NP, jnp.int32)
        for k in range(8):
            psrc[pl.ds(cnt + k * 16, 16)] = off16
            pdst[pl.ds(cnt + k * 16, 16)] = jnp.zeros((16,), jnp.int32)
            pex[pl.ds(cnt + k * 16, 16)] = zero16f

        @pl.when(cnt > 0)
        def _():
            flush(ftoff)

        plsc.subcore_barrier()

        for t in range(NSUB):
            ns = n0 + t * NROW
            pltpu.sync_copy(accum_sh.at[pl.ds(ns, NROW)], rows_v.at[0])
            pltpu.sync_copy(denom_sh.at[pl.ds(ns, NROW)], dnm2)
            pltpu.sync_copy(res_hbm.at[pl.ds(hh * NP + ns, NROW)],
                            rows_v.at[1])

            @plsc.parallel_loop(0, NROW, 1, unroll=4)
            def norm_body(r):
                dr = dnm2[r, :]
                d0 = jnp.maximum(jnp.full((16,), dr[0], jnp.float32), 1e-16)
                i0 = 1.0 / d0
                for jv in range(4):
                    x = rows_v[0, r, pl.ds(jv * 16, 16)] * i0 \
                        + rows_v[1, r, pl.ds(jv * 16, 16)]
                    outb_v[r, pl.ds(jv * 16, 16)] = _elu16(x)
            pltpu.sync_copy(outb_v, out_hbm.at[pl.ds(hh * NP + ns, NROW)])

        @plsc.parallel_loop(0, 128, 1, unroll=8)
        def zd2(i):
            dnm2[i, :] = jnp.zeros((16,), jnp.float32)

        plsc.subcore_barrier()


def _sce_body(pflat_hbm, tp_hbm, out_hbm, idx_v, idxb_v, bufa_v, outb_v, sem):
    cc = lax.axis_index("c")
    ss = lax.axis_index("s")
    wid = ss * CORES + cc
    pltpu.sync_copy(tp_hbm.at[pl.ds(wid * 32, 32)], idx_v)
    for r in range(32):
        for jv in range(4):
            outb_v[r, pl.ds(jv * 16, 16)] = jnp.zeros((16,), jnp.float32)
    for q in range(HEADS):
        for v in range(2):
            idxb_v[pl.ds(v * 16, 16)] = idx_v[pl.ds(v * 16, 16)] + q * NP
        pltpu.async_copy(pflat_hbm.at[idxb_v], bufa_v, sem).wait()
        for r in range(32):
            for jv in range(4):
                outb_v[r, pl.ds(jv * 16, 16)] = (
                    outb_v[r, pl.ds(jv * 16, 16)]
                    + bufa_v[r, pl.ds(jv * 16, 16)])
    pltpu.sync_copy(outb_v, out_hbm.at[pl.ds(wid * 32, 32)])


def _stack_w(ps, key):
    return jnp.stack([p[key] for p in ps])


def _stack_b(ps, key, n):
    return jnp.stack([p[key].reshape(1, n) for p in ps])


def _stack_wa(ps):
    wa = jnp.stack([jnp.concatenate([p['wl'], p['wr']], axis=1) for p in ps])
    ba = jnp.stack([jnp.stack([p['bl'][0], p['br'][0]]).reshape(1, 2)
                    for p in ps])
    return wa, ba


@jax.jit
def _run(features, edge_index, train_pad, params):
    f32 = jnp.float32
    l0, l1 = params['l0'], params['l1']
    w0 = _stack_w(l0, 'W')
    b0 = _stack_b(l0, 'b', 64)
    wa0, ba0 = _stack_wa(l0)
    w1 = _stack_w(l1, 'W')
    b1 = _stack_b(l1, 'b', 64)
    wa1, ba1 = _stack_wa(l1)
    wres = _stack_w(l1, 'Wres')
    bres = _stack_b(l1, 'bres', 64)

    xpad = jnp.pad(features, ((0, NP - N), (0, 0)))
    src3d = edge_index[0].reshape(NCH, 2, 128)
    dst3d = edge_index[1].reshape(NCH, 2, 128)
    ed3d = jnp.concatenate([src3d, dst3d], axis=1)
    zrow = jnp.zeros((NP, 64), f32)
    zdnm = jnp.zeros((NP, 16), f32)

    ft0, a12_0 = pl.pallas_call(
        _tc0_body,
        grid=(NP // BR, HEADS),
        in_specs=[
            pl.BlockSpec((BR, 128), lambda i, h: (i, 0)),
            pl.BlockSpec((1, 128, 64), lambda i, h: (h, 0, 0)),
            pl.BlockSpec((1, 1, 64), lambda i, h: (h, 0, 0)),
            pl.BlockSpec((1, 64, 2), lambda i, h: (h, 0, 0)),
            pl.BlockSpec((1, 1, 2), lambda i, h: (h, 0, 0)),
        ],
        out_specs=[
            pl.BlockSpec((BR, 64), lambda i, h: (h * (NP // BR) + i, 0)),
            pl.BlockSpec((1, BR, 2), lambda i, h: (h, i, 0)),
        ],
        out_shape=[
            jax.ShapeDtypeStruct((HEADS * NP, 64), f32),
            jax.ShapeDtypeStruct((HEADS, NP, 2), f32),
        ],
    )(xpad, w0, b0, wa0, ba0)

    sc0 = pl.kernel(
        _sc0_body,
        out_type=jax.ShapeDtypeStruct((HEADS * NP, 64), f32),
        mesh=_MESH,
        compiler_params=_SC_PARAMS,
        scratch_types=[
            pltpu.VMEM((2 * NP,), f32),
            pltpu.VMEM((2, 4, 128), jnp.int32),
            pltpu.VMEM((2, 2, 128), jnp.int32),
            pltpu.VMEM((2, 2, 128), jnp.int32),
            pltpu.VMEM((2, 256, 64), f32),
            pltpu.VMEM((2, 256, 16), f32),
            pltpu.SemaphoreType.DMA,
            pltpu.SemaphoreType.DMA,
            pltpu.SemaphoreType.DMA,
            pltpu.VMEM_SHARED((NP, 64), f32),
            pltpu.VMEM_SHARED((NP, 16), f32),
        ],
    )
    last = sc0(ft0, a12_0.reshape(HEADS, 2 * NP), ed3d, zrow, zdnm)

    ft1, a12_1, res1 = pl.pallas_call(
        _tc1_body,
        grid=(NP // BR, HEADS),
        in_specs=[
            pl.BlockSpec((BR, 64), lambda i, h: (0 * (NP // BR) + i, 0)),
            pl.BlockSpec((BR, 64), lambda i, h: (1 * (NP // BR) + i, 0)),
            pl.BlockSpec((BR, 64), lambda i, h: (2 * (NP // BR) + i, 0)),
            pl.BlockSpec((BR, 64), lambda i, h: (3 * (NP // BR) + i, 0)),
            pl.BlockSpec((1, 256, 64), lambda i, h: (h, 0, 0)),
            pl.BlockSpec((1, 1, 64), lambda i, h: (h, 0, 0)),
            pl.BlockSpec((1, 64, 2), lambda i, h: (h, 0, 0)),
            pl.BlockSpec((1, 1, 2), lambda i, h: (h, 0, 0)),
            pl.BlockSpec((1, 256, 64), lambda i, h: (h, 0, 0)),
            pl.BlockSpec((1, 1, 64), lambda i, h: (h, 0, 0)),
        ],
        out_specs=[
            pl.BlockSpec((BR, 64), lambda i, h: (h * (NP // BR) + i, 0)),
            pl.BlockSpec((1, BR, 2), lambda i, h: (h, i, 0)),
            pl.BlockSpec((BR, 64), lambda i, h: (h * (NP // BR) + i, 0)),
        ],
        out_shape=[
            jax.ShapeDtypeStruct((HEADS * NP, 64), f32),
            jax.ShapeDtypeStruct((HEADS, NP, 2), f32),
            jax.ShapeDtypeStruct((HEADS * NP, 64), f32),
        ],
    )(last, last, last, last, w1, b1, wa1, ba1, wres, bres)

    sc1 = pl.kernel(
        _sc1_body,
        out_type=jax.ShapeDtypeStruct((HEADS * NP, 64), f32),
        mesh=_MESH,
        compiler_params=_SC_PARAMS,
        scratch_types=[
            pltpu.VMEM((2 * NP,), f32),
            pltpu.VMEM((2, 4, 128), jnp.int32),
            pltpu.VMEM((NP,), jnp.int32),
            pltpu.VMEM((1024,), jnp.int32),
            pltpu.VMEM((528,), jnp.int32),
            pltpu.VMEM((528,), jnp.int32),
            pltpu.VMEM((528,), f32),
            pltpu.VMEM((1, 128), jnp.int32),
            pltpu.VMEM((1, 128), jnp.int32),
            pltpu.VMEM((2, 128, 64), f32),
            pltpu.VMEM((128, 16), f32),
            pltpu.VMEM((128, 64), f32),
            pltpu.SemaphoreType.DMA,
            pltpu.SemaphoreType.DMA,
            pltpu.SemaphoreType.DMA,
            pltpu.VMEM_SHARED((NP, 64), f32),
            pltpu.VMEM_SHARED((NP, 16), f32),
        ],
    )
    partial = sc1(ft1, a12_1.reshape(HEADS, 2 * NP),
                  ed3d, zrow, zdnm, res1, train_pad)

    sce = pl.kernel(
        _sce_body,
        out_type=jax.ShapeDtypeStruct((1024, 64), f32),
        mesh=_MESH,
        compiler_params=_SC_PARAMS,
        scratch_types=[
            pltpu.VMEM((32,), jnp.int32),
            pltpu.VMEM((32,), jnp.int32),
            pltpu.VMEM((32, 64), f32),
            pltpu.VMEM((32, 64), f32),
            pltpu.SemaphoreType.DMA,
        ],
    )
    outp = sce(partial, train_pad)
    return outp


def kernel(features, edge_index, train_nodes, params):
    train_pad = jnp.concatenate(
        [train_nodes, jnp.zeros((24,), jnp.int32)])
    outp = _run(features, edge_index, train_pad, params)
    return outp[:1000]
